# Initial kernel scaffold; baseline (speedup 1.0000x reference)
import jax
import jax.numpy as jnp
from jax.experimental import pallas as pl


def kernel(x, W, b):
    raise NotImplementedError("write your pallas kernel here")


# speedup vs baseline: 3.9878x; 3.9878x over previous
import functools

import jax
import jax.numpy as jnp
from jax import lax
from jax.experimental import pallas as pl
from jax.experimental.pallas import tpu as pltpu
from jax.experimental.pallas import tpu_sc as plsc

KNN = 32
BIGF = 3.0e38
INFL = 1.0e30
_INTERP = False


def _pick_rows(n):
    for r in (200, 128, 64, 32, 16, 8):
        if n % r == 0:
            return r
    return n


def _knn_body(xb_ref, xt_ref, sq_ref, out_ref, *, rows, np_, nch, gcols):
    pid = pl.program_id(0)
    xb = xb_ref[...]
    d2 = sq_ref[0:1, :] - 2.0 * jnp.dot(xb, xt_ref[...],
                                        preferred_element_type=jnp.float32)
    row_i = lax.broadcasted_iota(jnp.int32, (rows, np_), 0)
    col_i = lax.broadcasted_iota(jnp.int32, (rows, np_), 1)
    d2 = jnp.where(col_i == pid * rows + row_i, INFL, d2)

    gm = d2
    for s in (1, 2, 4, 8):
        gm = jnp.minimum(gm, pltpu.roll(gm, np_ - s, 1))

    lane = lax.broadcasted_iota(jnp.int32, (rows, 128), 1)
    stride_idx = 16 * (lane % 8)
    gmin_cols = []
    for v in range(gcols):
        acc = jnp.full((rows, 128), BIGF, jnp.float32)
        for ci in range(16):
            c = 16 * v + ci
            if c >= nch:
                break
            src = gm[:, c * 128:(c + 1) * 128]
            got = jnp.take_along_axis(src, stride_idx, axis=1)
            acc = jnp.where(lane // 8 == ci, got, acc)
        gmin_cols.append(acc)

    vmin = gmin_cols[0]
    for c in range(1, gcols):
        vmin = jnp.minimum(vmin, gmin_cols[c])
    gs = []
    for _ in range(KNN):
        m = jnp.min(vmin, axis=1, keepdims=True)
        lstar = jnp.min(jnp.where(vmin == m, lane, 999), axis=1, keepdims=True)
        cstar = jnp.full((rows, 1), 999, jnp.int32)
        for c in range(gcols - 1, -1, -1):
            vc = jnp.take_along_axis(gmin_cols[c], lstar, axis=1)
            cstar = jnp.where(vc == m, c, cstar)
        gs.append(cstar * 128 + lstar)
        for c in range(gcols):
            gmin_cols[c] = jnp.where((lane == lstar) & (cstar == c), BIGF,
                                     gmin_cols[c])
        vmin = gmin_cols[0]
        for c in range(1, gcols):
            vmin = jnp.minimum(vmin, gmin_cols[c])

    g128 = jnp.concatenate(gs * 4, axis=1)
    slot = lax.broadcasted_iota(jnp.int32, (rows, 512), 1)
    gsl = jnp.take_along_axis(g128, slot // 16, axis=1)
    candcol = gsl * 16 + slot % 16
    chunk = candcol // 128
    idxloc = candcol % 128
    cand = jnp.full((rows, 512), BIGF, jnp.float32)
    for c in range(nch):
        got = jnp.take_along_axis(d2[:, c * 128:(c + 1) * 128], idxloc, axis=1)
        cand = jnp.where(chunk == c, got, cand)

    outs = []
    for _ in range(KNN):
        m = jnp.min(cand, axis=1, keepdims=True)
        cc = jnp.min(jnp.where(cand == m, candcol, jnp.int32(2 ** 30)),
                     axis=1, keepdims=True)
        outs.append(cc)
        cand = jnp.where((cand == m) & (candcol == cc), BIGF, cand)
    out_ref[...] = jnp.concatenate(outs, axis=1)


def _knn_indices(x):
    n, d = x.shape
    rows = _pick_rows(n)
    np_ = -(-n // 128) * 128
    nch = np_ // 128
    gcols = -(-(np_ // 16) // 128)
    xp = jnp.pad(x, ((0, np_ - n), (0, 0)))
    sq = jnp.sum(xp * xp, axis=1)
    sq = jnp.where(lax.iota(jnp.int32, np_) < n, sq, INFL)
    sq8 = jnp.broadcast_to(sq[None, :], (8, np_))
    body = functools.partial(_knn_body, rows=rows, np_=np_, nch=nch,
                             gcols=gcols)
    return pl.pallas_call(
        body,
        grid=(n // rows,),
        in_specs=[
            pl.BlockSpec((rows, d), lambda i: (i, 0)),
            pl.BlockSpec((d, np_), lambda i: (0, 0)),
            pl.BlockSpec((8, np_), lambda i: (0, 0)),
        ],
        out_specs=pl.BlockSpec((rows, KNN), lambda i: (i, 0)),
        out_shape=jax.ShapeDtypeStruct((n, KNN), jnp.int32),
        compiler_params=pltpu.CompilerParams(
            dimension_semantics=("arbitrary",)),
        interpret=_INTERP,
    )(x, xp.T, sq8)


def _proj_body(x_ref, w_ref, b_ref, o_ref):
    o_ref[...] = (jnp.dot(x_ref[...], w_ref[...],
                          preferred_element_type=jnp.float32)
                  + b_ref[0:1, :])


def _projections(x, W, b):
    n, d = x.shape
    oc = W.shape[1]
    w1, w2 = W[:d], W[d:]
    wcat = jnp.concatenate([w1 - w2, w2], axis=1)
    bcat = jnp.concatenate([b, jnp.zeros_like(b)])
    b8 = jnp.broadcast_to(bcat[None, :], (8, 2 * oc))
    rows = _pick_rows(n)
    c = pl.pallas_call(
        _proj_body,
        grid=(n // rows,),
        in_specs=[
            pl.BlockSpec((rows, d), lambda i: (i, 0)),
            pl.BlockSpec((d, 2 * oc), lambda i: (0, 0)),
            pl.BlockSpec((8, 2 * oc), lambda i: (0, 0)),
        ],
        out_specs=pl.BlockSpec((rows, 2 * oc), lambda i: (i, 0)),
        out_shape=jax.ShapeDtypeStruct((n, 2 * oc), jnp.float32),
        compiler_params=pltpu.CompilerParams(
            dimension_semantics=("parallel",)),
        interpret=_INTERP,
    )(x, wcat, b8)
    return c[:, :oc], c[:, oc:]


def _gather_max(idx, bmat, amat, oc):
    info = plsc.get_sparse_core_info()
    nw = info.num_cores * info.num_subcores
    npad = idx.shape[0]
    per_w = npad // nw
    nb = 4
    nbat = per_w // nb
    idxf = idx.reshape(-1)
    mesh = plsc.VectorSubcoreMesh(core_axis_name="c", subcore_axis_name="s")

    @functools.partial(
        pl.kernel, mesh=mesh,
        out_type=jax.ShapeDtypeStruct((npad, oc), jnp.float32),
        scratch_types=[
            pltpu.VMEM((nb * KNN,), jnp.int32),
            pltpu.VMEM((nb * KNN, oc), jnp.float32),
            pltpu.VMEM((nb, oc), jnp.float32),
            pltpu.VMEM((nb, oc), jnp.float32),
            pltpu.SemaphoreType.DMA,
        ],
    )
    def k(idx_hbm, b_hbm, a_hbm, out_hbm, idx_v, rows_v, a_v, o_v, sem):
        wid = lax.axis_index("s") * info.num_cores + lax.axis_index("c")
        base = wid * per_w

        def batch_body(t, _):
            n0 = base + t * nb
            pltpu.sync_copy(idx_hbm.at[pl.ds(n0 * KNN, nb * KNN)], idx_v)
            pltpu.async_copy(b_hbm.at[idx_v], rows_v, sem).wait()
            pltpu.sync_copy(a_hbm.at[pl.ds(n0, nb)], a_v)

            def node_body(j, _):
                def vreg_body(v, _):
                    def red_body(r, acc):
                        return jnp.maximum(
                            acc, rows_v[j * KNN + r, pl.ds(v * 16, 16)])
                    acc = lax.fori_loop(
                        0, KNN, red_body,
                        jnp.full((16,), -BIGF, jnp.float32))
                    o_v[j, pl.ds(v * 16, 16)] = jnp.maximum(
                        acc + a_v[j, pl.ds(v * 16, 16)], 0.0)
                    return 0
                lax.fori_loop(0, oc // 16, vreg_body, 0)
                return 0
            lax.fori_loop(0, nb, node_body, 0)
            pltpu.sync_copy(o_v, out_hbm.at[pl.ds(n0, nb)])
            return 0

        lax.fori_loop(0, nbat, batch_body, 0)

    return k(idxf, bmat, amat)


def kernel(x, W, b):
    n, d = x.shape
    oc = W.shape[1]
    idx = _knn_indices(x)
    a, bm = _projections(x, W, b)
    npad = -(-n // 256) * 256
    idxp = jnp.pad(idx, ((0, npad - n), (0, 0)))
    ap = jnp.pad(a, ((0, npad - n), (0, 0)))
    out = _gather_max(idxp, bm, ap, oc)
    return out[:n]


# speedup vs baseline: 4.3200x; 1.0833x over previous
import functools

import jax
import jax.numpy as jnp
from jax import lax
from jax.experimental import pallas as pl
from jax.experimental.pallas import tpu as pltpu
from jax.experimental.pallas import tpu_sc as plsc

KNN = 32
BIGF = 3.0e38
INFL = 1.0e30
_INTERP = False


def _pick_rows(n):
    for r in (200, 128, 64, 32, 16, 8):
        if n % r == 0:
            return r
    return n


def _knn_body(xb_ref, xt_ref, sq_ref, out_ref, *, rows, np_, nch, gcols,
              offset):
    pid = pl.program_id(0)
    xb = xb_ref[...]
    d2 = sq_ref[0:1, :] - 2.0 * jnp.dot(xb, xt_ref[...],
                                        preferred_element_type=jnp.float32)
    row_i = lax.broadcasted_iota(jnp.int32, (rows, np_), 0)
    col_i = lax.broadcasted_iota(jnp.int32, (rows, np_), 1)
    d2 = jnp.where(col_i == offset + pid * rows + row_i, INFL, d2)

    gm = d2
    for s in (1, 2, 4, 8):
        gm = jnp.minimum(gm, pltpu.roll(gm, np_ - s, 1))

    lane = lax.broadcasted_iota(jnp.int32, (rows, 128), 1)
    stride_idx = 16 * (lane % 8)
    gmin_cols = []
    for v in range(gcols):
        acc = jnp.full((rows, 128), BIGF, jnp.float32)
        for ci in range(16):
            c = 16 * v + ci
            if c >= nch:
                break
            src = gm[:, c * 128:(c + 1) * 128]
            got = jnp.take_along_axis(src, stride_idx, axis=1)
            acc = jnp.where(lane // 8 == ci, got, acc)
        gmin_cols.append(acc)

    vmin = gmin_cols[0]
    for c in range(1, gcols):
        vmin = jnp.minimum(vmin, gmin_cols[c])
    gs = []
    for _ in range(KNN):
        m = jnp.min(vmin, axis=1, keepdims=True)
        lstar = jnp.min(jnp.where(vmin == m, lane, 999), axis=1, keepdims=True)
        cstar = jnp.full((rows, 1), 999, jnp.int32)
        for c in range(gcols - 1, -1, -1):
            vc = jnp.take_along_axis(gmin_cols[c], lstar, axis=1)
            cstar = jnp.where(vc == m, c, cstar)
        gs.append(cstar * 128 + lstar)
        for c in range(gcols):
            gmin_cols[c] = jnp.where((lane == lstar) & (cstar == c), BIGF,
                                     gmin_cols[c])
        vmin = gmin_cols[0]
        for c in range(1, gcols):
            vmin = jnp.minimum(vmin, gmin_cols[c])

    g128 = jnp.concatenate(gs * 4, axis=1)
    slot = lax.broadcasted_iota(jnp.int32, (rows, 512), 1)
    gsl = jnp.take_along_axis(g128, slot // 16, axis=1)
    candcol = gsl * 16 + slot % 16
    chunk = candcol // 128
    idxloc = candcol % 128
    cand = jnp.full((rows, 512), BIGF, jnp.float32)
    for c in range(nch):
        got = jnp.take_along_axis(d2[:, c * 128:(c + 1) * 128], idxloc, axis=1)
        cand = jnp.where(chunk == c, got, cand)

    outs = []
    for _ in range(KNN):
        m = jnp.min(cand, axis=1, keepdims=True)
        cc = jnp.min(jnp.where(cand == m, candcol, jnp.int32(2 ** 30)),
                     axis=1, keepdims=True)
        outs.append(cc)
        cand = jnp.where((cand == m) & (candcol == cc), BIGF, cand)
    out_ref[...] = jnp.concatenate(outs, axis=1)


def _knn_indices(xrows, xt, sq8, offset):
    nr, d = xrows.shape
    np_ = xt.shape[1]
    nch = np_ // 128
    gcols = -(-(np_ // 16) // 128)
    rows = _pick_rows(nr)
    body = functools.partial(_knn_body, rows=rows, np_=np_, nch=nch,
                             gcols=gcols, offset=offset)
    return pl.pallas_call(
        body,
        grid=(nr // rows,),
        in_specs=[
            pl.BlockSpec((rows, d), lambda i: (i, 0)),
            pl.BlockSpec((d, np_), lambda i: (0, 0)),
            pl.BlockSpec((8, np_), lambda i: (0, 0)),
        ],
        out_specs=pl.BlockSpec((rows, KNN), lambda i: (i, 0)),
        out_shape=jax.ShapeDtypeStruct((nr, KNN), jnp.int32),
        compiler_params=pltpu.CompilerParams(
            dimension_semantics=("arbitrary",)),
        interpret=_INTERP,
    )(xrows, xt, sq8)


def _proj_body(x_ref, w_ref, b_ref, o_ref):
    o_ref[...] = (jnp.dot(x_ref[...], w_ref[...],
                          preferred_element_type=jnp.float32)
                  + b_ref[0:1, :])


def _projections(x, W, b):
    n, d = x.shape
    oc = W.shape[1]
    w1, w2 = W[:d], W[d:]
    wcat = jnp.concatenate([w1 - w2, w2], axis=1)
    bcat = jnp.concatenate([b, jnp.zeros_like(b)])
    b8 = jnp.broadcast_to(bcat[None, :], (8, 2 * oc))
    rows = _pick_rows(n)
    c = pl.pallas_call(
        _proj_body,
        grid=(n // rows,),
        in_specs=[
            pl.BlockSpec((rows, d), lambda i: (i, 0)),
            pl.BlockSpec((d, 2 * oc), lambda i: (0, 0)),
            pl.BlockSpec((8, 2 * oc), lambda i: (0, 0)),
        ],
        out_specs=pl.BlockSpec((rows, 2 * oc), lambda i: (i, 0)),
        out_shape=jax.ShapeDtypeStruct((n, 2 * oc), jnp.float32),
        compiler_params=pltpu.CompilerParams(
            dimension_semantics=("parallel",)),
        interpret=_INTERP,
    )(x, wcat, b8)
    return c[:, :oc], c[:, oc:]


def _gather_max(idx, bmat, amat, oc):
    info = plsc.get_sparse_core_info()
    nw = info.num_cores * info.num_subcores
    npad = idx.shape[0]
    per_w = npad // nw
    nb = 4
    nbat = per_w // nb
    idxf = idx.reshape(-1)
    mesh = plsc.VectorSubcoreMesh(core_axis_name="c", subcore_axis_name="s")

    @functools.partial(
        pl.kernel, mesh=mesh,
        out_type=jax.ShapeDtypeStruct((npad, oc), jnp.float32),
        scratch_types=[
            pltpu.VMEM((nb * KNN,), jnp.int32),
            pltpu.VMEM((nb * KNN, oc), jnp.float32),
            pltpu.VMEM((nb, oc), jnp.float32),
            pltpu.VMEM((nb, oc), jnp.float32),
            pltpu.SemaphoreType.DMA,
        ],
    )
    def k(idx_hbm, b_hbm, a_hbm, out_hbm, idx_v, rows_v, a_v, o_v, sem):
        wid = lax.axis_index("s") * info.num_cores + lax.axis_index("c")
        base = wid * per_w

        def batch_body(t, _):
            n0 = base + t * nb
            pltpu.sync_copy(idx_hbm.at[pl.ds(n0 * KNN, nb * KNN)], idx_v)
            pltpu.async_copy(b_hbm.at[idx_v], rows_v, sem).wait()
            pltpu.sync_copy(a_hbm.at[pl.ds(n0, nb)], a_v)

            def node_body(j, _):
                def vreg_body(v, _):
                    def red_body(r, acc):
                        return jnp.maximum(
                            acc, rows_v[j * KNN + r, pl.ds(v * 16, 16)])
                    acc = lax.fori_loop(
                        0, KNN, red_body,
                        jnp.full((16,), -BIGF, jnp.float32))
                    o_v[j, pl.ds(v * 16, 16)] = jnp.maximum(
                        acc + a_v[j, pl.ds(v * 16, 16)], 0.0)
                    return 0
                lax.fori_loop(0, oc // 16, vreg_body, 0)
                return 0
            lax.fori_loop(0, nb, node_body, 0)
            pltpu.sync_copy(o_v, out_hbm.at[pl.ds(n0, nb)])
            return 0

        lax.fori_loop(0, nbat, batch_body, 0)

    return k(idxf, bmat, amat)


def kernel(x, W, b):
    n, d = x.shape
    oc = W.shape[1]
    np_ = -(-n // 128) * 128
    xp = jnp.pad(x, ((0, np_ - n), (0, 0)))
    sq = jnp.sum(xp * xp, axis=1)
    sq = jnp.where(lax.iota(jnp.int32, np_) < n, sq, INFL)
    sq8 = jnp.broadcast_to(sq[None, :], (8, np_))
    xt = xp.T
    a, bm = _projections(x, W, b)

    nparts = 5 if n % 5 == 0 else 1
    part = n // nparts
    outs = []
    for p in range(nparts):
        idx = _knn_indices(lax.slice(x, (p * part, 0), ((p + 1) * part, d)),
                           xt, sq8, p * part)
        npad = -(-part // 128) * 128
        idxp = jnp.pad(idx, ((0, npad - part), (0, 0)))
        ap = jnp.pad(lax.slice(a, (p * part, 0), ((p + 1) * part, oc)),
                     ((0, npad - part), (0, 0)))
        outs.append(_gather_max(idxp, bm, ap, oc)[:part])
    return jnp.concatenate(outs, axis=0)
